# baseline (speedup 1.0000x reference)
import jax
import jax.numpy as jnp
from jax.experimental import pallas as pl


def _assign_body(x_ref, c_ref, cn_ref, out_ref):
    xb = x_ref[...]
    s = jnp.dot(xb, c_ref[...], preferred_element_type=jnp.float32)
    xnorm = jnp.sum(xb * xb, axis=1, keepdims=True)
    dist = (xnorm + s) + cn_ref[...]

    R = dist.shape[0]
    v = dist[:, 0:128]
    j = jnp.zeros((R, 128), jnp.float32)
    for c in (1, 2, 3):
        vc = dist[:, c * 128:(c + 1) * 128]
        jc = jnp.full((R, 128), float(c * 128), jnp.float32)
        take = vc < v
        v = jnp.where(take, vc, v)
        j = jnp.where(take, jc, j)
    lane = jax.lax.broadcasted_iota(jnp.int32, (R, 128), 1).astype(jnp.float32)
    j = j + lane
    vt = v.T
    jt = j.T
    s = 128
    while s > 1:
        h = s // 2
        va, vb = vt[:h], vt[h:s]
        ja, jb = jt[:h], jt[h:s]
        take_b = (vb < va) | ((vb == va) & (jb < ja))
        vt = jnp.where(take_b, vb, va)
        jt = jnp.where(take_b, jb, ja)
        s = h
    out_ref[...] = jt[0].astype(jnp.int32)


def kernel(x, C, Cnorm):
    batched = x.ndim == 3
    x2 = x.reshape(-1, x.shape[-1]) if batched else x
    N, D = x2.shape
    K = C.shape[1]
    R = 2048 if N % 2048 == 0 else N
    out = pl.pallas_call(
        _assign_body,
        grid=(N // R,),
        in_specs=[
            pl.BlockSpec((R, D), lambda i: (i, 0)),
            pl.BlockSpec((D, K), lambda i: (0, 0)),
            pl.BlockSpec((1, K), lambda i: (0, 0)),
        ],
        out_specs=pl.BlockSpec((R,), lambda i: (i,)),
        out_shape=jax.ShapeDtypeStruct((N,), jnp.int32),
    )(x2, C * (-2.0), Cnorm)
    return out.reshape(x.shape[:-1]) if batched else out


# speedup vs baseline: 1.0756x; 1.0756x over previous
import jax
import jax.numpy as jnp
from jax.experimental import pallas as pl


def _assign_body(x_ref, c_ref, cn_ref, out_ref):
    xb = x_ref[...]
    s = jnp.dot(xb, c_ref[...] * (-2.0), preferred_element_type=jnp.float32)
    xnorm = jnp.sum(xb * xb, axis=1, keepdims=True)
    dist = (xnorm + s) + cn_ref[...]

    R = dist.shape[0]
    v = dist[:, 0:128]
    j = jnp.zeros((R, 128), jnp.float32)
    for c in (1, 2, 3):
        vc = dist[:, c * 128:(c + 1) * 128]
        jc = jnp.full((R, 128), float(c * 128), jnp.float32)
        take = vc < v
        v = jnp.where(take, vc, v)
        j = jnp.where(take, jc, j)
    lane = jax.lax.broadcasted_iota(jnp.int32, (R, 128), 1).astype(jnp.float32)
    j = j + lane
    vt = v.T
    jt = j.T
    s = 128
    while s > 1:
        h = s // 2
        va, vb = vt[:h], vt[h:s]
        ja, jb = jt[:h], jt[h:s]
        take_b = (vb < va) | ((vb == va) & (jb < ja))
        vt = jnp.where(take_b, vb, va)
        jt = jnp.where(take_b, jb, ja)
        s = h
    out_ref[...] = jt[0].astype(jnp.int32)


def kernel(x, C, Cnorm):
    batched = x.ndim == 3
    x2 = x.reshape(-1, x.shape[-1]) if batched else x
    N, D = x2.shape
    K = C.shape[1]
    R = 2048 if N % 2048 == 0 else N
    out = pl.pallas_call(
        _assign_body,
        grid=(N // R,),
        in_specs=[
            pl.BlockSpec((R, D), lambda i: (i, 0)),
            pl.BlockSpec((D, K), lambda i: (0, 0)),
            pl.BlockSpec((1, K), lambda i: (0, 0)),
        ],
        out_specs=pl.BlockSpec((R,), lambda i: (i,)),
        out_shape=jax.ShapeDtypeStruct((N,), jnp.int32),
    )(x2, C, Cnorm)
    return out.reshape(x.shape[:-1]) if batched else out
